# baseline (speedup 1.0000x reference)
import jax
import jax.numpy as jnp
from jax.experimental import pallas as pl
from jax.experimental.pallas import tpu as pltpu

N = 10000
D_IN = 128
HID = 64
ENC = 64
B = 4
T = 12
BN = B * N

ADJ_BLK = 400
NADJ = N // ADJ_BLK

BLK = 2000
NBLK = BN // BLK
NB_PER_B = N // BLK
HB = BLK // 2


def _extract_kernel(adj_ref, h_ref, ew1t_ref, eb1_ref, eps_ref,
                    bnw_ref, bnb_ref, ew2t_ref, eb2_ref, wg_ref, bg_ref,
                    wlt2_ref, bl_ref, ftc_ref,
                    x_scr, xs_scr, x2_scr, stats_scr):
    i = pl.program_id(0)

    @pl.when(i == 0)
    def _init():
        x = jnp.dot(h_ref[...], ew1t_ref[...],
                    preferred_element_type=jnp.float32) + eb1_ref[...]
        x_scr[...] = x
        xs_scr[:, :HID] = x.astype(jnp.bfloat16)
        xs_scr[:, HID:HID + 1] = jnp.ones((N, 1), jnp.bfloat16)
        xs_scr[:, HID + 1:] = jnp.zeros((N, D_IN - HID - 1), jnp.bfloat16)
        stats_scr[...] = jnp.zeros_like(stats_scr)

    a = adj_ref[...]
    po = jnp.dot(a.astype(jnp.bfloat16), xs_scr[...],
                 preferred_element_type=jnp.float32)
    pooled = po[:, :HID]
    degree = po[:, HID:HID + 1]
    degree = jnp.where(degree < 1e-6, jnp.float32(1.0), degree)
    xb = x_scr[pl.ds(i * ADJ_BLK, ADJ_BLK), :]
    x2 = pooled / degree + eps_ref[0] * xb
    x2_scr[pl.ds(i * ADJ_BLK, ADJ_BLK), :] = x2
    stats_scr[0:1, :] += jnp.sum(x2, axis=0, keepdims=True)
    stats_scr[1:2, :] += jnp.sum(x2 * x2, axis=0, keepdims=True)

    @pl.when(i == NADJ - 1)
    def _feat():
        mean = stats_scr[0:1, :] / N
        var = stats_scr[1:2, :] / N - mean * mean
        rstd = jax.lax.rsqrt(var + 1e-5)
        xn = (x2_scr[...] - mean) * rstd * bnw_ref[...] + bnb_ref[...]
        feat = jnp.dot(xn, ew2t_ref[...],
                       preferred_element_type=jnp.float32) + eb2_ref[...]
        ft = jnp.dot(feat, wg_ref[...],
                     preferred_element_type=jnp.float32) + bg_ref[0]
        ftc_ref[...] = jnp.dot(ft, wlt2_ref[...],
                               preferred_element_type=jnp.float32) \
            + bl_ref[...]


def _gru_kernel(inp_ref, inp_eo_ref, ftc_ref, w1h_ref, w1r_ref, b1_ref,
                w2h_ref, w2r_ref, b2_ref, wlt_ref, o_ref, a_scr):
    t = pl.program_id(0)
    bf16 = jnp.bfloat16
    f32 = jnp.float32

    @pl.when(t == 0)
    def _zero():
        o_ref[...] = jnp.zeros((BN, HID), f32)

    H2 = N // 2
    ftce = ftc_ref[pl.ds(0, 1), :, :].reshape(H2, HID)
    ftco = ftc_ref[pl.ds(1, 1), :, :].reshape(H2, HID)
    for b in range(B):
        h_b = o_ref[pl.ds(b * N, N), :]
        ir = inp_ref[pl.ds(t, 1), pl.ds(b, 1), :].reshape(1, N)
        irb = ir.astype(bf16)
        z = jnp.dot(h_b.astype(bf16), w1h_ref[...].astype(bf16),
                    preferred_element_type=f32)
        z = z + jax.lax.dot_general(
            irb, w1r_ref[...].astype(bf16),
            (((0,), (0,)), ((), ())), preferred_element_type=f32)
        a_scr[...] = jax.nn.sigmoid((z + b1_ref[...]).astype(bf16))

        ar = a_scr[pl.ds(0, H2), :]
        au = a_scr[pl.ds(H2, H2), :]
        he = o_ref[b * N + 0:(b + 1) * N:2, :]
        ho = o_ref[b * N + 1:(b + 1) * N:2, :]
        rh_e = he.astype(bf16) * ar[:, :HID]
        rh_o = ho.astype(bf16) * ar[:, HID:]
        rh = jnp.concatenate([rh_e, rh_o], axis=0)
        z2 = jnp.dot(rh, w2h_ref[...].astype(bf16),
                     preferred_element_type=f32)
        iro = inp_eo_ref[pl.ds(t, 1), pl.ds(b, 1), :, :].reshape(2, H2)
        w2rb = w2r_ref[...].astype(bf16)
        ze_in = jax.lax.dot_general(
            iro[0:1, :].astype(bf16), w2rb,
            (((0,), (0,)), ((), ())), preferred_element_type=f32)
        zo_in = jax.lax.dot_general(
            iro[1:2, :].astype(bf16), w2rb,
            (((0,), (0,)), ((), ())), preferred_element_type=f32)
        z2 = z2 + jnp.concatenate([ze_in, zo_in], axis=0)
        c = jnp.tanh((z2 + b2_ref[...]).astype(bf16))
        ue = au[:, :HID].astype(f32)
        uo = au[:, HID:].astype(f32)
        nh_e = ue * he + (1.0 - ue) * c[0:H2, :].astype(f32)
        nh_o = uo * ho + (1.0 - uo) * c[H2:, :].astype(f32)
        nh = jnp.concatenate([nh_e, nh_o], axis=0).astype(bf16)
        hn = jnp.dot(nh, wlt_ref[...].astype(bf16),
                     preferred_element_type=f32)
        o_ref[b * N + 0:(b + 1) * N:2, :] = hn[0:H2, :] + ftce
        o_ref[b * N + 1:(b + 1) * N:2, :] = hn[H2:, :] + ftco


@jax.jit
def kernel(h, adj, inputs, ew1, eb1, ew2, eb2, bnw, bnb, eps1,
           w1, b1, w2, b2, wg, bg, wl, bl):
    f32 = jnp.float32
    const2 = lambda i: (0, 0)

    ftc = pl.pallas_call(
        _extract_kernel,
        grid=(NADJ,),
        out_shape=jax.ShapeDtypeStruct((N, HID), f32),
        in_specs=[pl.BlockSpec((ADJ_BLK, N), lambda i: (i, 0)),
                  pl.BlockSpec((N, D_IN), const2),
                  pl.BlockSpec((D_IN, HID), const2),
                  pl.BlockSpec((1, HID), const2),
                  pl.BlockSpec(memory_space=pltpu.SMEM),
                  pl.BlockSpec((1, HID), const2),
                  pl.BlockSpec((1, HID), const2),
                  pl.BlockSpec((HID, ENC), const2),
                  pl.BlockSpec((1, ENC), const2),
                  pl.BlockSpec((ENC, ENC), const2),
                  pl.BlockSpec(memory_space=pltpu.SMEM),
                  pl.BlockSpec((ENC, HID), const2),
                  pl.BlockSpec((1, HID), const2)],
        out_specs=pl.BlockSpec((N, HID), const2),
        scratch_shapes=[pltpu.VMEM((N, HID), f32),
                        pltpu.VMEM((N, D_IN), jnp.bfloat16),
                        pltpu.VMEM((N, HID), f32),
                        pltpu.VMEM((8, HID), f32)],
    )(adj, h, ew1.T, eb1[None, :], eps1, bnw[None, :], bnb[None, :],
      ew2.T, eb2[None, :], wg, bg, wl[:, HID:].T, bl[None, :])

    inp_tm = inputs.transpose(1, 0, 2)
    inp_eo = inputs.reshape(B, T, N // 2, 2).transpose(1, 0, 3, 2)
    ftc_eo = ftc.reshape(N // 2, 2, HID).transpose(1, 0, 2)

    gconst2 = lambda t: (0, 0)
    gconst3 = lambda t: (0, 0, 0)
    gconst4 = lambda t: (0, 0, 0, 0)
    out = pl.pallas_call(
        _gru_kernel,
        grid=(T,),
        out_shape=jax.ShapeDtypeStruct((BN, HID), f32),
        in_specs=[pl.BlockSpec((T, B, N), gconst3),
                  pl.BlockSpec((T, B, 2, N // 2), gconst4),
                  pl.BlockSpec((2, N // 2, HID), gconst3),
                  pl.BlockSpec((HID, 2 * HID), gconst2),
                  pl.BlockSpec((1, 2 * HID), gconst2),
                  pl.BlockSpec((1, 2 * HID), gconst2),
                  pl.BlockSpec((HID, HID), gconst2),
                  pl.BlockSpec((1, HID), gconst2),
                  pl.BlockSpec((1, HID), gconst2),
                  pl.BlockSpec((HID, HID), gconst2)],
        out_specs=pl.BlockSpec((BN, HID), gconst2),
        scratch_shapes=[pltpu.VMEM((N, 2 * HID), jnp.bfloat16)],
    )(inp_tm, inp_eo, ftc_eo, w1[1:, :], w1[0:1, :], b1[None, :],
      w2[1:, :], w2[0:1, :], b2[None, :], wl[:, :HID].T)

    return out.reshape(B, N, HID)


# speedup vs baseline: 1.0633x; 1.0633x over previous
import jax
import jax.numpy as jnp
from jax.experimental import pallas as pl
from jax.experimental.pallas import tpu as pltpu

N = 10000
D_IN = 128
HID = 64
ENC = 64
B = 4
T = 12
BN = B * N

ADJ_BLK = 400
NADJ = N // ADJ_BLK

BLK = 2000
NBLK = BN // BLK
NB_PER_B = N // BLK
HB = BLK // 2


def _extract_kernel(adj_ref, h_ref, ew1t_ref, eb1_ref, eps_ref,
                    bnw_ref, bnb_ref, ew2t_ref, eb2_ref, wg_ref, bg_ref,
                    wlt2_ref, bl_ref, ftc_ref,
                    x_scr, xs_scr, x2_scr, stats_scr):
    i = pl.program_id(0)

    @pl.when(i == 0)
    def _init():
        x = jnp.dot(h_ref[...], ew1t_ref[...],
                    preferred_element_type=jnp.float32) + eb1_ref[...]
        x_scr[...] = x
        xs_scr[:, :HID] = x.astype(jnp.bfloat16)
        xs_scr[:, HID:HID + 1] = jnp.ones((N, 1), jnp.bfloat16)
        xs_scr[:, HID + 1:] = jnp.zeros((N, D_IN - HID - 1), jnp.bfloat16)
        stats_scr[...] = jnp.zeros_like(stats_scr)

    a = adj_ref[...]
    po = jnp.dot(a.astype(jnp.bfloat16), xs_scr[...],
                 preferred_element_type=jnp.float32)
    pooled = po[:, :HID]
    degree = po[:, HID:HID + 1]
    degree = jnp.where(degree < 1e-6, jnp.float32(1.0), degree)
    xb = x_scr[pl.ds(i * ADJ_BLK, ADJ_BLK), :]
    x2 = pooled / degree + eps_ref[0] * xb
    x2_scr[pl.ds(i * ADJ_BLK, ADJ_BLK), :] = x2
    stats_scr[0:1, :] += jnp.sum(x2, axis=0, keepdims=True)
    stats_scr[1:2, :] += jnp.sum(x2 * x2, axis=0, keepdims=True)

    @pl.when(i == NADJ - 1)
    def _feat():
        mean = stats_scr[0:1, :] / N
        var = stats_scr[1:2, :] / N - mean * mean
        rstd = jax.lax.rsqrt(var + 1e-5)
        xn = (x2_scr[...] - mean) * rstd * bnw_ref[...] + bnb_ref[...]
        feat = jnp.dot(xn, ew2t_ref[...],
                       preferred_element_type=jnp.float32) + eb2_ref[...]
        ft = jnp.dot(feat, wg_ref[...],
                     preferred_element_type=jnp.float32) + bg_ref[0]
        ftc_ref[...] = jnp.dot(ft, wlt2_ref[...],
                               preferred_element_type=jnp.float32) \
            + bl_ref[...]


def _gru_kernel(inp_ref, ftc_ref, w1h_ref, w1r_ref, b1_ref,
                w2h_ref, w2r_ref, b2_ref, wlt_ref, o_ref,
                a_scr, rg_scr, ug_scr):
    b = pl.program_id(0)
    t = pl.program_id(1)
    bf16 = jnp.bfloat16
    f32 = jnp.float32

    @pl.when(t == 0)
    def _zero():
        o_ref[...] = jnp.zeros((N, HID), f32)

    h_b = o_ref[...]
    ir = inp_ref[pl.ds(t, 1), pl.ds(b, 1), :].reshape(1, N)
    irb = ir.astype(bf16)
    z = jnp.dot(h_b.astype(bf16), w1h_ref[...].astype(bf16),
                preferred_element_type=f32)
    z = z + jax.lax.dot_general(
        irb, w1r_ref[...].astype(bf16),
        (((0,), (0,)), ((), ())), preferred_element_type=f32)
    a_scr[...] = jax.nn.sigmoid((z + b1_ref[...]).astype(bf16))

    ar = a_scr[pl.ds(0, N // 2), :].astype(f32)
    au = a_scr[pl.ds(N // 2, N // 2), :].astype(f32)
    rg_scr[0::2, :] = ar[:, :HID]
    rg_scr[1::2, :] = ar[:, HID:]
    ug_scr[0::2, :] = au[:, :HID]
    ug_scr[1::2, :] = au[:, HID:]

    rh = rg_scr[...] * h_b
    z2 = jnp.dot(rh.astype(bf16), w2h_ref[...].astype(bf16),
                 preferred_element_type=f32)
    z2 = z2 + jax.lax.dot_general(
        irb, w2r_ref[...].astype(bf16),
        (((0,), (0,)), ((), ())), preferred_element_type=f32)
    c = jnp.tanh((z2 + b2_ref[...]).astype(bf16)).astype(f32)
    u = ug_scr[...]
    nh = u * h_b + (1.0 - u) * c
    hn = jnp.dot(nh.astype(bf16), wlt_ref[...].astype(bf16),
                 preferred_element_type=f32)
    o_ref[...] = hn + ftc_ref[...]


@jax.jit
def kernel(h, adj, inputs, ew1, eb1, ew2, eb2, bnw, bnb, eps1,
           w1, b1, w2, b2, wg, bg, wl, bl):
    f32 = jnp.float32
    const2 = lambda i: (0, 0)

    ftc = pl.pallas_call(
        _extract_kernel,
        grid=(NADJ,),
        out_shape=jax.ShapeDtypeStruct((N, HID), f32),
        in_specs=[pl.BlockSpec((ADJ_BLK, N), lambda i: (i, 0)),
                  pl.BlockSpec((N, D_IN), const2),
                  pl.BlockSpec((D_IN, HID), const2),
                  pl.BlockSpec((1, HID), const2),
                  pl.BlockSpec(memory_space=pltpu.SMEM),
                  pl.BlockSpec((1, HID), const2),
                  pl.BlockSpec((1, HID), const2),
                  pl.BlockSpec((HID, ENC), const2),
                  pl.BlockSpec((1, ENC), const2),
                  pl.BlockSpec((ENC, ENC), const2),
                  pl.BlockSpec(memory_space=pltpu.SMEM),
                  pl.BlockSpec((ENC, HID), const2),
                  pl.BlockSpec((1, HID), const2)],
        out_specs=pl.BlockSpec((N, HID), const2),
        scratch_shapes=[pltpu.VMEM((N, HID), f32),
                        pltpu.VMEM((N, D_IN), jnp.bfloat16),
                        pltpu.VMEM((N, HID), f32),
                        pltpu.VMEM((8, HID), f32)],
    )(adj, h, ew1.T, eb1[None, :], eps1, bnw[None, :], bnb[None, :],
      ew2.T, eb2[None, :], wg, bg, wl[:, HID:].T, bl[None, :])

    inp_tm = inputs.transpose(1, 0, 2)

    gconst2 = lambda b, t: (0, 0)
    gconst3 = lambda b, t: (0, 0, 0)
    out = pl.pallas_call(
        _gru_kernel,
        grid=(B, T),
        out_shape=jax.ShapeDtypeStruct((BN, HID), f32),
        in_specs=[pl.BlockSpec((T, B, N), gconst3),
                  pl.BlockSpec((N, HID), gconst2),
                  pl.BlockSpec((HID, 2 * HID), gconst2),
                  pl.BlockSpec((1, 2 * HID), gconst2),
                  pl.BlockSpec((1, 2 * HID), gconst2),
                  pl.BlockSpec((HID, HID), gconst2),
                  pl.BlockSpec((1, HID), gconst2),
                  pl.BlockSpec((1, HID), gconst2),
                  pl.BlockSpec((HID, HID), gconst2)],
        out_specs=pl.BlockSpec((N, HID), lambda b, t: (b, 0)),
        scratch_shapes=[pltpu.VMEM((N, 2 * HID), jnp.bfloat16),
                        pltpu.VMEM((N, HID), f32),
                        pltpu.VMEM((N, HID), f32)],
        compiler_params=pltpu.CompilerParams(
            dimension_semantics=("parallel", "arbitrary")),
    )(inp_tm, ftc, w1[1:, :], w1[0:1, :], b1[None, :],
      w2[1:, :], w2[0:1, :], b2[None, :], wl[:, :HID].T)

    return out.reshape(B, N, HID)


# speedup vs baseline: 1.0790x; 1.0148x over previous
import jax
import jax.numpy as jnp
from jax.experimental import pallas as pl
from jax.experimental.pallas import tpu as pltpu

N = 10000
D_IN = 128
HID = 64
ENC = 64
B = 4
T = 12
BN = B * N

ADJ_BLK = 400
NADJ = N // ADJ_BLK

BLK = 2000
NBLK = BN // BLK
NB_PER_B = N // BLK
HB = BLK // 2


def _extract_kernel(adj_ref, h_ref, ew1t_ref, eb1_ref, eps_ref,
                    bnw_ref, bnb_ref, ew2t_ref, eb2_ref, wg_ref, bg_ref,
                    wlt2_ref, bl_ref, ftc_ref,
                    x_scr, xs_scr, x2_scr, stats_scr):
    i = pl.program_id(0)

    @pl.when(i == 0)
    def _init():
        x = jnp.dot(h_ref[...], ew1t_ref[...],
                    preferred_element_type=jnp.float32) + eb1_ref[...]
        x_scr[...] = x
        xs_scr[:, :HID] = x.astype(jnp.bfloat16)
        xs_scr[:, HID:HID + 1] = jnp.ones((N, 1), jnp.bfloat16)
        xs_scr[:, HID + 1:] = jnp.zeros((N, D_IN - HID - 1), jnp.bfloat16)
        stats_scr[...] = jnp.zeros_like(stats_scr)

    a = adj_ref[...]
    po = jnp.dot(a.astype(jnp.bfloat16), xs_scr[...],
                 preferred_element_type=jnp.float32)
    pooled = po[:, :HID]
    degree = po[:, HID:HID + 1]
    degree = jnp.where(degree < 1e-6, jnp.float32(1.0), degree)
    xb = x_scr[pl.ds(i * ADJ_BLK, ADJ_BLK), :]
    x2 = pooled / degree + eps_ref[0] * xb
    x2_scr[pl.ds(i * ADJ_BLK, ADJ_BLK), :] = x2
    stats_scr[0:1, :] += jnp.sum(x2, axis=0, keepdims=True)
    stats_scr[1:2, :] += jnp.sum(x2 * x2, axis=0, keepdims=True)

    @pl.when(i == NADJ - 1)
    def _feat():
        mean = stats_scr[0:1, :] / N
        var = stats_scr[1:2, :] / N - mean * mean
        rstd = jax.lax.rsqrt(var + 1e-5)
        xn = (x2_scr[...] - mean) * rstd * bnw_ref[...] + bnb_ref[...]
        feat = jnp.dot(xn, ew2t_ref[...],
                       preferred_element_type=jnp.float32) + eb2_ref[...]
        ft = jnp.dot(feat, wg_ref[...],
                     preferred_element_type=jnp.float32) + bg_ref[0]
        ftc_ref[...] = jnp.dot(ft, wlt2_ref[...],
                               preferred_element_type=jnp.float32) \
            + bl_ref[...]


def _gru_kernel(inp_ref, ftc_ref, w1h_ref, w1r_ref, b1_ref,
                w2h_ref, w2r_ref, b2_ref, wlt_ref, o_ref,
                a_scr, rg_scr, ug_scr):
    t = pl.program_id(0)
    bf16 = jnp.bfloat16
    f32 = jnp.float32

    @pl.when(t == 0)
    def _zero():
        o_ref[...] = jnp.zeros((BN, HID), f32)

    ftcv = ftc_ref[...]
    for b in range(B):
        h_b = o_ref[pl.ds(b * N, N), :]
        ir = inp_ref[pl.ds(t, 1), pl.ds(b, 1), :].reshape(1, N)
        irb = ir.astype(bf16)
        z = jnp.dot(h_b.astype(bf16), w1h_ref[...].astype(bf16),
                    preferred_element_type=f32)
        z = z + jax.lax.dot_general(
            irb, w1r_ref[...].astype(bf16),
            (((0,), (0,)), ((), ())), preferred_element_type=f32)
        a_scr[...] = jax.nn.sigmoid((z + b1_ref[...]).astype(bf16))

        ar = a_scr[pl.ds(0, N // 2), :].astype(f32)
        au = a_scr[pl.ds(N // 2, N // 2), :].astype(f32)
        rg_scr[0::2, :] = ar[:, :HID]
        rg_scr[1::2, :] = ar[:, HID:]
        ug_scr[0::2, :] = au[:, :HID]
        ug_scr[1::2, :] = au[:, HID:]

        rh = rg_scr[...] * h_b
        z2 = jnp.dot(rh.astype(bf16), w2h_ref[...].astype(bf16),
                     preferred_element_type=f32)
        z2 = z2 + jax.lax.dot_general(
            irb, w2r_ref[...].astype(bf16),
            (((0,), (0,)), ((), ())), preferred_element_type=f32)
        c = jnp.tanh((z2 + b2_ref[...]).astype(bf16)).astype(f32)
        u = ug_scr[...]
        nh = u * h_b + (1.0 - u) * c
        hn = jnp.dot(nh.astype(bf16), wlt_ref[...].astype(bf16),
                     preferred_element_type=f32)
        o_ref[pl.ds(b * N, N), :] = hn + ftcv


@jax.jit
def kernel(h, adj, inputs, ew1, eb1, ew2, eb2, bnw, bnb, eps1,
           w1, b1, w2, b2, wg, bg, wl, bl):
    f32 = jnp.float32
    const2 = lambda i: (0, 0)

    ftc = pl.pallas_call(
        _extract_kernel,
        grid=(NADJ,),
        out_shape=jax.ShapeDtypeStruct((N, HID), f32),
        in_specs=[pl.BlockSpec((ADJ_BLK, N), lambda i: (i, 0)),
                  pl.BlockSpec((N, D_IN), const2),
                  pl.BlockSpec((D_IN, HID), const2),
                  pl.BlockSpec((1, HID), const2),
                  pl.BlockSpec(memory_space=pltpu.SMEM),
                  pl.BlockSpec((1, HID), const2),
                  pl.BlockSpec((1, HID), const2),
                  pl.BlockSpec((HID, ENC), const2),
                  pl.BlockSpec((1, ENC), const2),
                  pl.BlockSpec((ENC, ENC), const2),
                  pl.BlockSpec(memory_space=pltpu.SMEM),
                  pl.BlockSpec((ENC, HID), const2),
                  pl.BlockSpec((1, HID), const2)],
        out_specs=pl.BlockSpec((N, HID), const2),
        scratch_shapes=[pltpu.VMEM((N, HID), f32),
                        pltpu.VMEM((N, D_IN), jnp.bfloat16),
                        pltpu.VMEM((N, HID), f32),
                        pltpu.VMEM((8, HID), f32)],
    )(adj, h, ew1.T, eb1[None, :], eps1, bnw[None, :], bnb[None, :],
      ew2.T, eb2[None, :], wg, bg, wl[:, HID:].T, bl[None, :])

    inp_tm = inputs.transpose(1, 0, 2)

    gconst2 = lambda t: (0, 0)
    gconst3 = lambda t: (0, 0, 0)
    out = pl.pallas_call(
        _gru_kernel,
        grid=(T,),
        out_shape=jax.ShapeDtypeStruct((BN, HID), f32),
        in_specs=[pl.BlockSpec((T, B, N), gconst3),
                  pl.BlockSpec((N, HID), gconst2),
                  pl.BlockSpec((HID, 2 * HID), gconst2),
                  pl.BlockSpec((1, 2 * HID), gconst2),
                  pl.BlockSpec((1, 2 * HID), gconst2),
                  pl.BlockSpec((HID, HID), gconst2),
                  pl.BlockSpec((1, HID), gconst2),
                  pl.BlockSpec((1, HID), gconst2),
                  pl.BlockSpec((HID, HID), gconst2)],
        out_specs=pl.BlockSpec((BN, HID), gconst2),
        scratch_shapes=[pltpu.VMEM((N, 2 * HID), jnp.bfloat16),
                        pltpu.VMEM((N, HID), f32),
                        pltpu.VMEM((N, HID), f32)],
    )(inp_tm, ftc, w1[1:, :], w1[0:1, :], b1[None, :],
      w2[1:, :], w2[0:1, :], b2[None, :], wl[:, :HID].T)

    return out.reshape(B, N, HID)


# speedup vs baseline: 1.1389x; 1.0555x over previous
import jax
import jax.numpy as jnp
from jax.experimental import pallas as pl
from jax.experimental.pallas import tpu as pltpu

N = 10000
D_IN = 128
HID = 64
ENC = 64
B = 4
T = 12
BN = B * N

ADJ_BLK = 400
NADJ = N // ADJ_BLK

BLK = 2000
NBLK = BN // BLK
NB_PER_B = N // BLK
HB = BLK // 2


def _extract_kernel(adj_ref, h_ref, ew1t_ref, eb1_ref, eps_ref,
                    bnw_ref, bnb_ref, ew2t_ref, eb2_ref, wg_ref, bg_ref,
                    wlt2_ref, bl_ref, ftc_ref,
                    x_scr, xs_scr, x2_scr, stats_scr):
    i = pl.program_id(0)

    @pl.when(i == 0)
    def _init():
        x = jnp.dot(h_ref[...], ew1t_ref[...],
                    preferred_element_type=jnp.float32) + eb1_ref[...]
        x_scr[...] = x
        xs_scr[:, :HID] = x.astype(jnp.bfloat16)
        xs_scr[:, HID:HID + 1] = jnp.ones((N, 1), jnp.bfloat16)
        xs_scr[:, HID + 1:] = jnp.zeros((N, D_IN - HID - 1), jnp.bfloat16)
        stats_scr[...] = jnp.zeros_like(stats_scr)

    a = adj_ref[...]
    po = jnp.dot(a.astype(jnp.bfloat16), xs_scr[...],
                 preferred_element_type=jnp.float32)
    pooled = po[:, :HID]
    degree = po[:, HID:HID + 1]
    degree = jnp.where(degree < 1e-6, jnp.float32(1.0), degree)
    xb = x_scr[pl.ds(i * ADJ_BLK, ADJ_BLK), :]
    x2 = pooled / degree + eps_ref[0] * xb
    x2_scr[pl.ds(i * ADJ_BLK, ADJ_BLK), :] = x2
    stats_scr[0:1, :] += jnp.sum(x2, axis=0, keepdims=True)
    stats_scr[1:2, :] += jnp.sum(x2 * x2, axis=0, keepdims=True)

    @pl.when(i == NADJ - 1)
    def _feat():
        mean = stats_scr[0:1, :] / N
        var = stats_scr[1:2, :] / N - mean * mean
        rstd = jax.lax.rsqrt(var + 1e-5)
        xn = (x2_scr[...] - mean) * rstd * bnw_ref[...] + bnb_ref[...]
        feat = jnp.dot(xn, ew2t_ref[...],
                       preferred_element_type=jnp.float32) + eb2_ref[...]
        ft = jnp.dot(feat, wg_ref[...],
                     preferred_element_type=jnp.float32) + bg_ref[0]
        ftc_ref[...] = jnp.dot(ft, wlt2_ref[...],
                               preferred_element_type=jnp.float32) \
            + bl_ref[...]


def _gru_kernel(inp_ref, ftc_ref, w1h_ref, w1r_ref, b1_ref,
                w2h_ref, w2r_ref, b2_ref, wlt_ref, o_ref,
                a_scr, rg_scr, ug_scr):
    t = pl.program_id(0)
    bf16 = jnp.bfloat16
    f32 = jnp.float32

    @pl.when(t == 0)
    def _zero():
        o_ref[...] = jnp.zeros((BN, HID), f32)

    ftcv = ftc_ref[...]
    for b in range(B):
        h_b = o_ref[pl.ds(b * N, N), :]
        ir = inp_ref[pl.ds(t, 1), pl.ds(b, 1), :].reshape(1, N)
        irb = ir.astype(bf16)
        z = jnp.dot(h_b.astype(bf16), w1h_ref[...].astype(bf16),
                    preferred_element_type=f32)
        z = z + jax.lax.dot_general(
            irb, w1r_ref[...].astype(bf16),
            (((0,), (0,)), ((), ())), preferred_element_type=f32)
        a_scr[...] = jax.nn.sigmoid(z + b1_ref[...]).astype(bf16)

        ar = a_scr[pl.ds(0, N // 2), :].astype(f32)
        au = a_scr[pl.ds(N // 2, N // 2), :].astype(f32)
        rg_scr[0::2, :] = ar[:, :HID]
        rg_scr[1::2, :] = ar[:, HID:]
        ug_scr[0::2, :] = au[:, :HID]
        ug_scr[1::2, :] = au[:, HID:]

        rh = rg_scr[...] * h_b
        z2 = jnp.dot(rh.astype(bf16), w2h_ref[...].astype(bf16),
                     preferred_element_type=f32)
        z2 = z2 + jax.lax.dot_general(
            irb, w2r_ref[...].astype(bf16),
            (((0,), (0,)), ((), ())), preferred_element_type=f32)
        c = jnp.tanh(z2 + b2_ref[...])
        u = ug_scr[...]
        nh = u * h_b + (1.0 - u) * c
        hn = jnp.dot(nh.astype(bf16), wlt_ref[...].astype(bf16),
                     preferred_element_type=f32)
        o_ref[pl.ds(b * N, N), :] = hn + ftcv


@jax.jit
def kernel(h, adj, inputs, ew1, eb1, ew2, eb2, bnw, bnb, eps1,
           w1, b1, w2, b2, wg, bg, wl, bl):
    f32 = jnp.float32
    const2 = lambda i: (0, 0)

    ftc = pl.pallas_call(
        _extract_kernel,
        grid=(NADJ,),
        out_shape=jax.ShapeDtypeStruct((N, HID), f32),
        in_specs=[pl.BlockSpec((ADJ_BLK, N), lambda i: (i, 0)),
                  pl.BlockSpec((N, D_IN), const2),
                  pl.BlockSpec((D_IN, HID), const2),
                  pl.BlockSpec((1, HID), const2),
                  pl.BlockSpec(memory_space=pltpu.SMEM),
                  pl.BlockSpec((1, HID), const2),
                  pl.BlockSpec((1, HID), const2),
                  pl.BlockSpec((HID, ENC), const2),
                  pl.BlockSpec((1, ENC), const2),
                  pl.BlockSpec((ENC, ENC), const2),
                  pl.BlockSpec(memory_space=pltpu.SMEM),
                  pl.BlockSpec((ENC, HID), const2),
                  pl.BlockSpec((1, HID), const2)],
        out_specs=pl.BlockSpec((N, HID), const2),
        scratch_shapes=[pltpu.VMEM((N, HID), f32),
                        pltpu.VMEM((N, D_IN), jnp.bfloat16),
                        pltpu.VMEM((N, HID), f32),
                        pltpu.VMEM((8, HID), f32)],
    )(adj, h, ew1.T, eb1[None, :], eps1, bnw[None, :], bnb[None, :],
      ew2.T, eb2[None, :], wg, bg, wl[:, HID:].T, bl[None, :])

    inp_tm = inputs.transpose(1, 0, 2)

    gconst2 = lambda t: (0, 0)
    gconst3 = lambda t: (0, 0, 0)
    out = pl.pallas_call(
        _gru_kernel,
        grid=(T,),
        out_shape=jax.ShapeDtypeStruct((BN, HID), f32),
        in_specs=[pl.BlockSpec((T, B, N), gconst3),
                  pl.BlockSpec((N, HID), gconst2),
                  pl.BlockSpec((HID, 2 * HID), gconst2),
                  pl.BlockSpec((1, 2 * HID), gconst2),
                  pl.BlockSpec((1, 2 * HID), gconst2),
                  pl.BlockSpec((HID, HID), gconst2),
                  pl.BlockSpec((1, HID), gconst2),
                  pl.BlockSpec((1, HID), gconst2),
                  pl.BlockSpec((HID, HID), gconst2)],
        out_specs=pl.BlockSpec((BN, HID), gconst2),
        scratch_shapes=[pltpu.VMEM((N, 2 * HID), jnp.bfloat16),
                        pltpu.VMEM((N, HID), f32),
                        pltpu.VMEM((N, HID), f32)],
    )(inp_tm, ftc, w1[1:, :], w1[0:1, :], b1[None, :],
      w2[1:, :], w2[0:1, :], b2[None, :], wl[:, :HID].T)

    return out.reshape(B, N, HID)
